# baseline (speedup 1.0000x reference)
import functools

import jax
import jax.numpy as jnp
from jax import lax
from jax.experimental import pallas as pl
from jax.experimental.pallas import tpu as pltpu
from jax.experimental.pallas import tpu_sc as plsc

_TAU = 5.0
_K = 4096
_TL = 1024


def _logits_body(tok_ref, cond_ref, w_ref, b_ref, out_ref):
    t = tok_ref[...] + cond_ref[0]
    tb = t.astype(jnp.bfloat16).astype(jnp.float32)
    wb = w_ref[...].astype(jnp.bfloat16).astype(jnp.float32)
    out_ref[...] = jnp.sum(tb * wb, axis=1, keepdims=True) + b_ref[0]


def _q_body(logits_ref, g_ref, q_ref):
    s = (logits_ref[...] + g_ref[...]) / _TAU
    m = jnp.max(s, axis=1, keepdims=True)
    y = jnp.exp(s - m)
    q_ref[...] = y / jnp.sum(y, axis=1, keepdims=True)


def _make_sc_select(B, L):
    mesh = plsc.VectorSubcoreMesh(core_axis_name="c", subcore_axis_name="s")
    nch = L // 16

    @functools.partial(
        pl.kernel,
        mesh=mesh,
        out_type=jax.ShapeDtypeStruct((B, L), jnp.float32),
        compiler_params=pltpu.CompilerParams(needs_layout_passes=False),
        scratch_types=[
            pltpu.VMEM((L,), jnp.float32),
            pltpu.VMEM((L,), jnp.float32),
            pltpu.VMEM((L + 16,), jnp.int32),
            pltpu.VMEM((L + 16,), jnp.int32),
        ],
    )
    def sc_select(q_hbm, mask_hbm, qv, mv, va, vb):
        wid = lax.axis_index("s") * 2 + lax.axis_index("c")

        @pl.when(wid < B)
        def _():
            row = wid
            pltpu.sync_copy(q_hbm.at[row], qv)
            iota16 = lax.iota(jnp.int32, 16)
            ones16 = jnp.ones((16,), jnp.int32)
            kvec = jnp.full((16,), _K, jnp.int32)

            def init(i, carry):
                va[pl.ds(i * 16, 16)] = lax.bitcast_convert_type(
                    qv[pl.ds(i * 16, 16)], jnp.int32)
                return carry

            lax.fori_loop(0, nch, init, jnp.int32(0))

            bufs = [(va, vb), (vb, va)]
            n = jnp.int32(L)
            count_above = jnp.zeros((16,), jnp.int32)
            prefix = jnp.zeros((16,), jnp.int32)
            for bi in range(30):
                bit = 29 - bi
                src_v, dst_v = bufs[bi % 2]
                nch_d = (n + 15) // 16

                def cbody(i, acc):
                    v = src_v[pl.ds(i * 16, 16)]
                    hi = ((v >> bit) & 1) == 1
                    valid = (iota16 + i * 16) < n
                    return acc + plsc.all_reduce_population_count(hi & valid)

                c = lax.fori_loop(0, nch_d, cbody, jnp.zeros((16,), jnp.int32))
                take_hi = (count_above + c) >= kvec

                def kbody(i, off):
                    v = src_v[pl.ds(i * 16, 16)]
                    hi = ((v >> bit) & 1) == 1
                    keep = jnp.where(take_hi, hi, ~hi)
                    keep = keep & ((iota16 + i * 16) < n)
                    plsc.store_compressed(dst_v.at[pl.ds(off, 16)], v, mask=keep)
                    return off + plsc.all_reduce_population_count(keep)[0]

                n2 = lax.fori_loop(0, nch_d, kbody, jnp.int32(0))
                dst_v[pl.ds(n2, 16)] = jnp.zeros((16,), jnp.int32)
                prefix = jnp.where(take_hi, prefix | (1 << bit), prefix)
                count_above = jnp.where(take_hi, count_above, count_above + c)
                n = n2

            uk = prefix
            need = kvec - count_above

            def wbody(i, tiecnt):
                v = lax.bitcast_convert_type(qv[pl.ds(i * 16, 16)], jnp.int32)
                gt = v > uk
                eq = v == uk
                run = plsc.cumsum(ones16, mask=eq)
                sel = gt | (eq & ((tiecnt + run) <= need))
                mv[pl.ds(i * 16, 16)] = jnp.where(sel, 1.0, 0.0).astype(jnp.float32)
                return tiecnt + plsc.all_reduce_population_count(eq)

            lax.fori_loop(0, nch, wbody, jnp.zeros((16,), jnp.int32))
            pltpu.sync_copy(mv, mask_hbm.at[row])

    return sc_select


def kernel(token, cond, W, b):
    B, L, D = token.shape
    g = jax.random.gumbel(jax.random.key(42), (B, L), jnp.float32)

    tok2 = token.reshape(B * L, D)
    cond3 = cond.reshape(B, 1, D)
    blocks_per_batch = L // _TL
    logits = pl.pallas_call(
        _logits_body,
        grid=(B * L // _TL,),
        in_specs=[
            pl.BlockSpec((_TL, D), lambda j: (j, 0)),
            pl.BlockSpec((1, 1, D), lambda j: (j // blocks_per_batch, 0, 0)),
            pl.BlockSpec((1, D), lambda j: (0, 0)),
            pl.BlockSpec(memory_space=pltpu.SMEM),
        ],
        out_specs=pl.BlockSpec((_TL, 1), lambda j: (j, 0)),
        out_shape=jax.ShapeDtypeStruct((B * L, 1), jnp.float32),
    )(tok2, cond3, W, b)
    logits = logits.reshape(B, L)

    q = pl.pallas_call(
        _q_body,
        out_shape=jax.ShapeDtypeStruct((B, L), jnp.float32),
    )(logits, g)

    mask = _make_sc_select(B, L)(q)

    return (mask, logits)


# speedup vs baseline: 1.3488x; 1.3488x over previous
import functools

import jax
import jax.numpy as jnp
from jax import lax
from jax.experimental import pallas as pl
from jax.experimental.pallas import tpu as pltpu
from jax.experimental.pallas import tpu_sc as plsc

_TAU = 5.0
_K = 4096
_TL = 1024


def _logits_body(tok_ref, cond_ref, w_ref, b_ref, out_ref):
    t = tok_ref[...] + cond_ref[0]
    tb = t.astype(jnp.bfloat16).astype(jnp.float32)
    wb = w_ref[...].astype(jnp.bfloat16).astype(jnp.float32)
    out_ref[...] = jnp.sum(tb * wb, axis=1, keepdims=True) + b_ref[0]


def _q_body(logits_ref, g_ref, q_ref):
    s = (logits_ref[...] + g_ref[...]) / _TAU
    m = jnp.max(s, axis=1, keepdims=True)
    y = jnp.exp(s - m)
    q_ref[...] = y / jnp.sum(y, axis=1, keepdims=True)


def _make_sc_select(B, L):
    mesh = plsc.VectorSubcoreMesh(core_axis_name="c", subcore_axis_name="s")
    nch = L // 16

    @functools.partial(
        pl.kernel,
        mesh=mesh,
        out_type=jax.ShapeDtypeStruct((B, L), jnp.float32),
        compiler_params=pltpu.CompilerParams(needs_layout_passes=False),
        scratch_types=[
            pltpu.VMEM((L,), jnp.float32),
            pltpu.VMEM((L,), jnp.float32),
            pltpu.VMEM((L + 16,), jnp.int32),
            pltpu.VMEM((L + 16,), jnp.int32),
            pltpu.VMEM((256,), jnp.int32),
        ],
    )
    def sc_select(q_hbm, mask_hbm, qv, mv, va, vb, hist):
        wid = lax.axis_index("s") * 2 + lax.axis_index("c")

        @pl.when(wid < B)
        def _():
            row = wid
            pltpu.sync_copy(q_hbm.at[row], qv)
            iota16 = lax.iota(jnp.int32, 16)
            ones16 = jnp.ones((16,), jnp.int32)
            kvec = jnp.full((16,), _K, jnp.int32)

            n = jnp.int32(L)
            count_above = jnp.zeros((16,), jnp.int32)
            uk = jnp.zeros((16,), jnp.int32)
            levels = [(22, 256, None, va), (14, 256, va, vb),
                      (6, 256, vb, va), (0, 64, va, vb)]
            for shift, nb, src, dst in levels:
                maskb = nb - 1

                def zbody(i, c):
                    hist[pl.ds(i * 16, 16)] = jnp.zeros((16,), jnp.int32)
                    return c

                lax.fori_loop(0, nb // 16, zbody, jnp.int32(0))
                nch_d = (n + 15) // 16

                def read(i):
                    if src is None:
                        return lax.bitcast_convert_type(
                            qv[pl.ds(i * 16, 16)], jnp.int32)
                    return src[pl.ds(i * 16, 16)]

                def hbody(i, c):
                    idx = (read(i) >> shift) & maskb
                    valid = (iota16 + i * 16) < n
                    plsc.addupdate_scatter(hist, [idx], ones16, mask=valid)
                    return c

                lax.fori_loop(0, nch_d, hbody, jnp.int32(0))

                thr = count_above + n - _K

                def sbody(i, carry):
                    ps, b_star, found = carry
                    pc = plsc.cumsum(hist[pl.ds(i * 16, 16)])
                    f = plsc.all_reduce_ffs((ps + pc) > thr)
                    hit = f < 16
                    b_star = jnp.where((found > 0) | (~hit), b_star, i * 16 + f)
                    return (ps + pc[15], b_star, found | hit.astype(jnp.int32))

                _, b_star, _ = lax.fori_loop(
                    0, nb // 16, sbody,
                    (jnp.zeros((16,), jnp.int32), jnp.zeros((16,), jnp.int32),
                     jnp.zeros((16,), jnp.int32)))

                def cbody(i, carry):
                    off, ca = carry
                    idx = (read(i) >> shift) & maskb
                    valid = (iota16 + i * 16) < n
                    gtb = (idx > b_star) & valid
                    eqb = (idx == b_star) & valid
                    plsc.store_compressed(
                        dst.at[pl.ds(off, 16)], read(i), mask=eqb)
                    return (off + plsc.all_reduce_population_count(eqb)[0],
                            ca + plsc.all_reduce_population_count(gtb))

                n, count_above = lax.fori_loop(
                    0, nch_d, cbody,
                    (jnp.int32(0), count_above))
                uk = uk | (b_star << shift)

            need = kvec - count_above

            def wbody(i, tiecnt):
                v = lax.bitcast_convert_type(qv[pl.ds(i * 16, 16)], jnp.int32)
                gt = v > uk
                eq = v == uk
                run = plsc.cumsum(ones16, mask=eq)
                sel = gt | (eq & ((tiecnt + run) <= need))
                mv[pl.ds(i * 16, 16)] = jnp.where(sel, 1.0, 0.0).astype(jnp.float32)
                return tiecnt + plsc.all_reduce_population_count(eq)

            lax.fori_loop(0, nch, wbody, jnp.zeros((16,), jnp.int32))
            pltpu.sync_copy(mv, mask_hbm.at[row])

    return sc_select


def kernel(token, cond, W, b):
    B, L, D = token.shape
    g = jax.random.gumbel(jax.random.key(42), (B, L), jnp.float32)

    tok2 = token.reshape(B * L, D)
    cond3 = cond.reshape(B, 1, D)
    blocks_per_batch = L // _TL
    logits = pl.pallas_call(
        _logits_body,
        grid=(B * L // _TL,),
        in_specs=[
            pl.BlockSpec((_TL, D), lambda j: (j, 0)),
            pl.BlockSpec((1, 1, D), lambda j: (j // blocks_per_batch, 0, 0)),
            pl.BlockSpec((1, D), lambda j: (0, 0)),
            pl.BlockSpec(memory_space=pltpu.SMEM),
        ],
        out_specs=pl.BlockSpec((_TL, 1), lambda j: (j, 0)),
        out_shape=jax.ShapeDtypeStruct((B * L, 1), jnp.float32),
    )(tok2, cond3, W, b)
    logits = logits.reshape(B, L)

    q = pl.pallas_call(
        _q_body,
        out_shape=jax.ShapeDtypeStruct((B, L), jnp.float32),
    )(logits, g)

    mask = _make_sc_select(B, L)(q)

    return (mask, logits)


# speedup vs baseline: 1.4568x; 1.0801x over previous
import functools

import jax
import jax.numpy as jnp
from jax import lax
from jax.experimental import pallas as pl
from jax.experimental.pallas import tpu as pltpu
from jax.experimental.pallas import tpu_sc as plsc

_TAU = 5.0
_K = 4096
_TL = 1024


def _logits_body(tok_ref, cond_ref, w_ref, b_ref, out_ref):
    t = tok_ref[...] + cond_ref[0]
    tb = t.astype(jnp.bfloat16).astype(jnp.float32)
    wb = w_ref[...].astype(jnp.bfloat16).astype(jnp.float32)
    out_ref[...] = jnp.sum(tb * wb, axis=1, keepdims=True) + b_ref[0]


def _q_body(logits_ref, g_ref, q_ref):
    s = (logits_ref[...] + g_ref[...]) / _TAU
    m = jnp.max(s, axis=1, keepdims=True)
    y = jnp.exp(s - m)
    q_ref[...] = y / jnp.sum(y, axis=1, keepdims=True)


def _make_sc_select(B, L):
    mesh = plsc.VectorSubcoreMesh(core_axis_name="c", subcore_axis_name="s")
    nch = L // 16

    @functools.partial(
        pl.kernel,
        mesh=mesh,
        out_type=jax.ShapeDtypeStruct((B, L), jnp.float32),
        compiler_params=pltpu.CompilerParams(needs_layout_passes=False),
        scratch_types=[
            pltpu.VMEM((L,), jnp.float32),
            pltpu.VMEM((L,), jnp.float32),
            pltpu.VMEM((L + 16,), jnp.int32),
            pltpu.VMEM((L + 16,), jnp.int32),
            pltpu.VMEM((256,), jnp.int32),
        ],
    )
    def sc_select(q_hbm, mask_hbm, qv, mv, va, vb, hist):
        wid = lax.axis_index("s") * 2 + lax.axis_index("c")

        @pl.when(wid < B)
        def _():
            row = wid
            pltpu.sync_copy(q_hbm.at[row], qv)
            iota16 = lax.iota(jnp.int32, 16)
            ones16 = jnp.ones((16,), jnp.int32)
            kvec = jnp.full((16,), _K, jnp.int32)

            n = jnp.int32(L)
            count_above = jnp.zeros((16,), jnp.int32)
            uk = jnp.zeros((16,), jnp.int32)
            levels = [(22, 256, None, va), (14, 256, va, vb),
                      (6, 256, vb, va), (0, 64, va, vb)]
            for shift, nb, src, dst in levels:
                maskb = nb - 1

                @plsc.parallel_loop(0, nb // 16)
                def zbody(i):
                    hist[pl.ds(i * 16, 16)] = jnp.zeros((16,), jnp.int32)
                nch_d = (n + 15) // 16

                def read(i):
                    if src is None:
                        return lax.bitcast_convert_type(
                            qv[pl.ds(i * 16, 16)], jnp.int32)
                    return src[pl.ds(i * 16, 16)]

                @plsc.parallel_loop(0, nch_d, unroll=4)
                def hbody(i):
                    idx = (read(i) >> shift) & maskb
                    valid = (iota16 + i * 16) < n
                    plsc.addupdate_scatter(hist, [idx], ones16, mask=valid)

                thr = count_above + n - _K

                def sbody(i, carry):
                    ps, b_star, found = carry
                    pc = plsc.cumsum(hist[pl.ds(i * 16, 16)])
                    f = plsc.all_reduce_ffs((ps + pc) > thr)
                    hit = f < 16
                    b_star = jnp.where((found > 0) | (~hit), b_star, i * 16 + f)
                    return (ps + pc[15], b_star, found | hit.astype(jnp.int32))

                _, b_star, _ = plsc.parallel_loop(
                    0, nb // 16,
                    carry=(jnp.zeros((16,), jnp.int32),
                           jnp.zeros((16,), jnp.int32),
                           jnp.zeros((16,), jnp.int32)))(sbody)

                def cbody(i, carry):
                    off, ca = carry
                    idx = (read(i) >> shift) & maskb
                    valid = (iota16 + i * 16) < n
                    gtb = (idx > b_star) & valid
                    eqb = (idx == b_star) & valid
                    plsc.store_compressed(
                        dst.at[pl.ds(off, 16)], read(i), mask=eqb)
                    return (off + plsc.all_reduce_population_count(eqb)[0],
                            ca + plsc.all_reduce_population_count(gtb))

                n, count_above = plsc.parallel_loop(
                    0, nch_d, unroll=2,
                    carry=(jnp.int32(0), count_above))(cbody)
                uk = uk | (b_star << shift)

            need = kvec - count_above

            def wbody(i, tiecnt):
                v = lax.bitcast_convert_type(qv[pl.ds(i * 16, 16)], jnp.int32)
                gt = v > uk
                eq = v == uk
                run = plsc.cumsum(ones16, mask=eq)
                sel = gt | (eq & ((tiecnt + run) <= need))
                mv[pl.ds(i * 16, 16)] = jnp.where(sel, 1.0, 0.0).astype(jnp.float32)
                return tiecnt + plsc.all_reduce_population_count(eq)

            plsc.parallel_loop(
                0, nch, unroll=4,
                carry=jnp.zeros((16,), jnp.int32))(wbody)
            pltpu.sync_copy(mv, mask_hbm.at[row])

    return sc_select


def kernel(token, cond, W, b):
    B, L, D = token.shape
    g = jax.random.gumbel(jax.random.key(42), (B, L), jnp.float32)

    tok2 = token.reshape(B * L, D)
    cond3 = cond.reshape(B, 1, D)
    blocks_per_batch = L // _TL
    logits = pl.pallas_call(
        _logits_body,
        grid=(B * L // _TL,),
        in_specs=[
            pl.BlockSpec((_TL, D), lambda j: (j, 0)),
            pl.BlockSpec((1, 1, D), lambda j: (j // blocks_per_batch, 0, 0)),
            pl.BlockSpec((1, D), lambda j: (0, 0)),
            pl.BlockSpec(memory_space=pltpu.SMEM),
        ],
        out_specs=pl.BlockSpec((_TL, 1), lambda j: (j, 0)),
        out_shape=jax.ShapeDtypeStruct((B * L, 1), jnp.float32),
    )(tok2, cond3, W, b)
    logits = logits.reshape(B, L)

    q = pl.pallas_call(
        _q_body,
        out_shape=jax.ShapeDtypeStruct((B, L), jnp.float32),
    )(logits, g)

    mask = _make_sc_select(B, L)(q)

    return (mask, logits)


# speedup vs baseline: 1.4821x; 1.0174x over previous
import functools

import jax
import jax.numpy as jnp
from jax import lax
from jax.experimental import pallas as pl
from jax.experimental.pallas import tpu as pltpu
from jax.experimental.pallas import tpu_sc as plsc

_TAU = 5.0
_K = 4096
_TL = 1024


def _logits_body(tok_ref, cond_ref, w_ref, b_ref, out_ref):
    t = tok_ref[...] + cond_ref[0]
    tb = t.astype(jnp.bfloat16).astype(jnp.float32)
    wb = w_ref[...].astype(jnp.bfloat16).astype(jnp.float32)
    out_ref[...] = jnp.sum(tb * wb, axis=1, keepdims=True) + b_ref[0]


def _q_body(logits_ref, g_ref, q_ref):
    s = (logits_ref[...] + g_ref[...]) / _TAU
    m = jnp.max(s, axis=1, keepdims=True)
    y = jnp.exp(s - m)
    q_ref[...] = y / jnp.sum(y, axis=1, keepdims=True)


def _make_sc_select(B, L):
    mesh = plsc.VectorSubcoreMesh(core_axis_name="c", subcore_axis_name="s")
    nch = L // 16

    @functools.partial(
        pl.kernel,
        mesh=mesh,
        out_type=jax.ShapeDtypeStruct((B, L), jnp.float32),
        compiler_params=pltpu.CompilerParams(needs_layout_passes=False),
        scratch_types=[
            pltpu.VMEM((L,), jnp.float32),
            pltpu.VMEM((L,), jnp.float32),
            pltpu.VMEM((L + 16,), jnp.int32),
            pltpu.VMEM((L + 16,), jnp.int32),
            pltpu.VMEM((256,), jnp.int32),
        ],
    )
    def sc_select(q_hbm, mask_hbm, qv, mv, va, vb, hist):
        wid = lax.axis_index("s") * 2 + lax.axis_index("c")

        @pl.when(wid < B)
        def _():
            row = wid
            pltpu.sync_copy(q_hbm.at[row], qv)
            iota16 = lax.iota(jnp.int32, 16)
            ones16 = jnp.ones((16,), jnp.int32)
            kvec = jnp.full((16,), _K, jnp.int32)

            n = jnp.int32(L)
            count_above = jnp.zeros((16,), jnp.int32)
            uk = jnp.zeros((16,), jnp.int32)
            levels = [(22, 256, None, va), (14, 256, va, vb),
                      (6, 256, vb, va), (0, 64, va, vb)]
            for shift, nb, src, dst in levels:
                maskb = nb - 1

                @plsc.parallel_loop(0, nb // 16)
                def zbody(i):
                    hist[pl.ds(i * 16, 16)] = jnp.zeros((16,), jnp.int32)
                nch_d = (n + 15) // 16

                def read(i):
                    if src is None:
                        return lax.bitcast_convert_type(
                            qv[pl.ds(i * 16, 16)], jnp.int32)
                    return src[pl.ds(i * 16, 16)]

                @plsc.parallel_loop(0, nch_d, unroll=8)
                def hbody(i):
                    idx = (read(i) >> shift) & maskb
                    valid = (iota16 + i * 16) < n
                    plsc.addupdate_scatter(hist, [idx], ones16, mask=valid)

                thr = count_above + n - _K

                def sbody(i, carry):
                    ps, b_star, found = carry
                    pc = plsc.cumsum(hist[pl.ds(i * 16, 16)])
                    f = plsc.all_reduce_ffs((ps + pc) > thr)
                    hit = f < 16
                    b_star = jnp.where((found > 0) | (~hit), b_star, i * 16 + f)
                    return (ps + pc[15], b_star, found | hit.astype(jnp.int32))

                _, b_star, _ = plsc.parallel_loop(
                    0, nb // 16,
                    carry=(jnp.zeros((16,), jnp.int32),
                           jnp.zeros((16,), jnp.int32),
                           jnp.zeros((16,), jnp.int32)))(sbody)

                def cbody(i, carry):
                    off, ca = carry
                    v = read(i)
                    idx = (v >> shift) & maskb
                    valid = (iota16 + i * 16) < n
                    gtb = (idx > b_star) & valid
                    eqb = (idx == b_star) & valid
                    plsc.store_compressed(
                        dst.at[pl.ds(off, 16)], v, mask=eqb)
                    return (off + plsc.all_reduce_population_count(eqb)[0],
                            ca + plsc.all_reduce_population_count(gtb))

                n, count_above = plsc.parallel_loop(
                    0, nch_d, unroll=2,
                    carry=(jnp.int32(0), count_above))(cbody)
                uk = uk | (b_star << shift)

            need = kvec - count_above

            def wbody(i, tiecnt):
                v = lax.bitcast_convert_type(qv[pl.ds(i * 16, 16)], jnp.int32)
                gt = v > uk
                eq = v == uk
                run = plsc.cumsum(ones16, mask=eq)
                sel = gt | (eq & ((tiecnt + run) <= need))
                mv[pl.ds(i * 16, 16)] = jnp.where(sel, 1.0, 0.0).astype(jnp.float32)
                return tiecnt + plsc.all_reduce_population_count(eq)

            plsc.parallel_loop(
                0, nch, unroll=8,
                carry=jnp.zeros((16,), jnp.int32))(wbody)
            pltpu.sync_copy(mv, mask_hbm.at[row])

    return sc_select


def kernel(token, cond, W, b):
    B, L, D = token.shape
    g = jax.random.gumbel(jax.random.key(42), (B, L), jnp.float32)

    tok2 = token.reshape(B * L, D)
    cond3 = cond.reshape(B, 1, D)
    blocks_per_batch = L // _TL
    logits = pl.pallas_call(
        _logits_body,
        grid=(B * L // _TL,),
        in_specs=[
            pl.BlockSpec((_TL, D), lambda j: (j, 0)),
            pl.BlockSpec((1, 1, D), lambda j: (j // blocks_per_batch, 0, 0)),
            pl.BlockSpec((1, D), lambda j: (0, 0)),
            pl.BlockSpec(memory_space=pltpu.SMEM),
        ],
        out_specs=pl.BlockSpec((_TL, 1), lambda j: (j, 0)),
        out_shape=jax.ShapeDtypeStruct((B * L, 1), jnp.float32),
    )(tok2, cond3, W, b)
    logits = logits.reshape(B, L)

    q = pl.pallas_call(
        _q_body,
        out_shape=jax.ShapeDtypeStruct((B, L), jnp.float32),
    )(logits, g)

    mask = _make_sc_select(B, L)(q)

    return (mask, logits)
